# baseline (speedup 1.0000x reference)
import functools

import jax
import jax.numpy as jnp
from jax import lax
from jax.experimental import pallas as pl
from jax.experimental.pallas import tpu as pltpu
from jax.experimental.pallas import tpu_sc as plsc

_E = 16
_K = 2
_LANES = 16
_NC = 2
_NS = 16


def _tc_body(hs_ref, gate_ref, logits_ref, final_ref):
    x = hs_ref[...]
    g = gate_ref[...]
    logits_ref[...] = lax.dot_general(
        g, x, (((1,), (1,)), ((), ())),
        preferred_element_type=jnp.float32,
        precision=lax.Precision.DEFAULT,
    )
    final_ref[...] = jnp.zeros_like(final_ref)


def _sc_route_body(logits_hbm, mask_hbm, logits_v, mask_v):
    t = logits_hbm.shape[1]
    chunk = t // (_NC * _NS)
    wid = lax.axis_index("s") * _NC + lax.axis_index("c")
    base = wid * chunk
    pltpu.sync_copy(logits_hbm.at[:, pl.ds(base, chunk)], logits_v)

    def group(gi, _):
        sl = pl.ds(gi * _LANES, _LANES)
        m1 = logits_v[0, sl]
        i1 = jnp.zeros((_LANES,), jnp.int32)
        for e in range(1, _E):
            v = logits_v[e, sl]
            c = v > m1
            m1 = jnp.where(c, v, m1)
            i1 = jnp.where(c, jnp.full((_LANES,), e, jnp.int32), i1)
        m2 = jnp.full((_LANES,), -jnp.inf, jnp.float32)
        i2 = jnp.zeros((_LANES,), jnp.int32)
        for e in range(_E):
            v = logits_v[e, sl]
            c = jnp.logical_and(v > m2, i1 != e)
            m2 = jnp.where(c, v, m2)
            i2 = jnp.where(c, jnp.full((_LANES,), e, jnp.int32), i2)
        one = jnp.ones((_LANES,), jnp.int32)
        zero = jnp.zeros((_LANES,), jnp.int32)
        for e in range(_E):
            mask_v[e, 0, sl] = jnp.where(i1 == e, one, zero)
            mask_v[e, 1, sl] = jnp.where(i2 == e, one, zero)
        return 0

    lax.fori_loop(0, chunk // _LANES, group, 0)
    pltpu.sync_copy(mask_v, mask_hbm.at[:, :, pl.ds(base, chunk)])


def _sc_route(logits_t):
    t = logits_t.shape[1]
    chunk = t // (_NC * _NS)
    return pl.kernel(
        _sc_route_body,
        out_type=jax.ShapeDtypeStruct((_E, _K, t), jnp.int32),
        mesh=plsc.VectorSubcoreMesh(core_axis_name="c", subcore_axis_name="s"),
        scratch_types=[
            pltpu.VMEM((_E, chunk), jnp.float32),
            pltpu.VMEM((_E, _K, chunk), jnp.int32),
        ],
    )(logits_t)


def kernel(hidden_states, gate_w):
    batch, seq, d = hidden_states.shape
    t = batch * seq
    hs = hidden_states.reshape(t, d)
    bt = 512
    grid = t // bt
    logits_t, final = pl.pallas_call(
        _tc_body,
        grid=(grid,),
        in_specs=[
            pl.BlockSpec((bt, d), lambda i: (i, 0)),
            pl.BlockSpec((_E, d), lambda i: (0, 0)),
        ],
        out_specs=[
            pl.BlockSpec((_E, bt), lambda i: (0, i)),
            pl.BlockSpec((bt, d), lambda i: (i, 0)),
        ],
        out_shape=[
            jax.ShapeDtypeStruct((_E, t), jnp.float32),
            jax.ShapeDtypeStruct((t, d), jnp.float32),
        ],
    )(hs, gate_w)
    mask = _sc_route(logits_t)
    return final, mask


# speedup vs baseline: 1.0489x; 1.0489x over previous
import functools

import jax
import jax.numpy as jnp
from jax import lax
from jax.experimental import pallas as pl
from jax.experimental.pallas import tpu as pltpu
from jax.experimental.pallas import tpu_sc as plsc

_E = 16
_K = 2
_LANES = 16
_NC = 2
_NS = 16


def _tc_matmul_body(hs_ref, gate_ref, logits_ref):
    x = hs_ref[...]
    g = gate_ref[...]
    logits_ref[...] = lax.dot_general(
        g, x, (((1,), (1,)), ((), ())),
        preferred_element_type=jnp.float32,
        precision=lax.Precision.DEFAULT,
    )


def _tc_zeros_body(final_ref):
    final_ref[...] = jnp.zeros_like(final_ref)


def _sc_route_body(logits_hbm, mask_hbm, logits_v, mask_v):
    t = logits_hbm.shape[1]
    chunk = t // (_NC * _NS)
    wid = lax.axis_index("s") * _NC + lax.axis_index("c")
    base = wid * chunk
    pltpu.sync_copy(logits_hbm.at[:, pl.ds(base, chunk)], logits_v)

    def group(gi, _):
        sl = pl.ds(gi * _LANES, _LANES)
        m1 = logits_v[0, sl]
        i1 = jnp.zeros((_LANES,), jnp.int32)
        for e in range(1, _E):
            v = logits_v[e, sl]
            c = v > m1
            m1 = jnp.where(c, v, m1)
            i1 = jnp.where(c, jnp.full((_LANES,), e, jnp.int32), i1)
        m2 = jnp.full((_LANES,), -jnp.inf, jnp.float32)
        i2 = jnp.zeros((_LANES,), jnp.int32)
        for e in range(_E):
            v = logits_v[e, sl]
            c = jnp.logical_and(v > m2, i1 != e)
            m2 = jnp.where(c, v, m2)
            i2 = jnp.where(c, jnp.full((_LANES,), e, jnp.int32), i2)
        one = jnp.ones((_LANES,), jnp.int32)
        zero = jnp.zeros((_LANES,), jnp.int32)
        for e in range(_E):
            mask_v[e, 0, sl] = jnp.where(i1 == e, one, zero)
            mask_v[e, 1, sl] = jnp.where(i2 == e, one, zero)
        return 0

    lax.fori_loop(0, chunk // _LANES, group, 0)
    pltpu.sync_copy(mask_v, mask_hbm.at[:, :, pl.ds(base, chunk)])


def _sc_route(logits_t):
    t = logits_t.shape[1]
    chunk = t // (_NC * _NS)
    return pl.kernel(
        _sc_route_body,
        out_type=jax.ShapeDtypeStruct((_E, _K, t), jnp.int32),
        mesh=plsc.VectorSubcoreMesh(core_axis_name="c", subcore_axis_name="s"),
        scratch_types=[
            pltpu.VMEM((_E, chunk), jnp.float32),
            pltpu.VMEM((_E, _K, chunk), jnp.int32),
        ],
    )(logits_t)


def kernel(hidden_states, gate_w):
    batch, seq, d = hidden_states.shape
    t = batch * seq
    hs = hidden_states.reshape(t, d)
    bt = 512
    grid = t // bt
    logits_t = pl.pallas_call(
        _tc_matmul_body,
        grid=(grid,),
        in_specs=[
            pl.BlockSpec((bt, d), lambda i: (i, 0)),
            pl.BlockSpec((_E, d), lambda i: (0, 0)),
        ],
        out_specs=pl.BlockSpec((_E, bt), lambda i: (0, i)),
        out_shape=jax.ShapeDtypeStruct((_E, t), jnp.float32),
    )(hs, gate_w)
    mask = _sc_route(logits_t)
    final = pl.pallas_call(
        _tc_zeros_body,
        grid=(grid,),
        out_specs=pl.BlockSpec((bt, d), lambda i: (i, 0)),
        out_shape=jax.ShapeDtypeStruct((t, d), jnp.float32),
    )()
    return final, mask
